# baseline (speedup 1.0000x reference)
import functools

import jax
import jax.numpy as jnp
from jax import lax
from jax.experimental import pallas as pl
from jax.experimental.pallas import tpu as pltpu
from jax.experimental.pallas import tpu_sc as plsc

NC, NS = 2, 16
NW = NC * NS
BW = 128


@functools.partial(jax.jit, static_argnums=(4, 5, 6))
def _dual_gather(src_table, tgt_table, sidx_t, tidx_t, b, l, d):
    mesh = plsc.VectorSubcoreMesh(core_axis_name="c", subcore_axis_name="s")

    @functools.partial(
        pl.kernel,
        out_type=(
            jax.ShapeDtypeStruct((l, d // 8, NW, 8, BW), jnp.float32),
            jax.ShapeDtypeStruct((l, d // 8, NW, 8, BW), jnp.float32),
        ),
        mesh=mesh,
        scratch_types=[
            pltpu.VMEM((l, BW), jnp.int32),
            pltpu.VMEM((l, BW), jnp.int32),
            pltpu.VMEM((BW, d), jnp.float32),
            pltpu.VMEM((BW, d), jnp.float32),
            pltpu.VMEM((d // 8, 8, BW), jnp.float32),
            pltpu.VMEM((d // 8, 8, BW), jnp.float32),
            pltpu.SemaphoreType.DMA,
            pltpu.SemaphoreType.DMA,
            pltpu.SemaphoreType.DMA,
            pltpu.SemaphoreType.DMA,
        ],
        compiler_params=pltpu.CompilerParams(
            use_tc_tiling_on_sc=False, needs_layout_passes=False),
    )
    def k(src_tab, tgt_tab, sidx_hbm, tidx_hbm, src_out, tgt_out,
          sidx_v, tidx_v, gbuf0, gbuf1, tbuf0, tbuf1, g0, g1, w0, w1):
        wid = lax.axis_index("s") * NC + lax.axis_index("c")
        col0 = wid * BW
        pltpu.sync_copy(sidx_hbm.at[:, pl.ds(col0, BW)], sidx_v)
        pltpu.sync_copy(tidx_hbm.at[:, pl.ds(col0, BW)], tidx_v)

        lanes = lax.iota(jnp.int32, 16)

        rows16 = [lanes + i0 for i0 in range(0, BW, 16)]

        def transpose_tile(gb, tb):
            for e in range(d):
                col = jnp.full((16,), e, jnp.int32)
                for i, r in enumerate(rows16):
                    v = plsc.load_gather(gb, [r, col])
                    tb[e // 8, e % 8, pl.ds(i * 16, 16)] = v

        gbufs = (gbuf0, gbuf1)
        tbufs = (tbuf0, tbuf1)
        gsems = (g0, g1)
        wsems = (w0, w1)

        def phase(tab, idx_v, out):
            def fire_gather(li, s):
                pltpu.async_copy(tab.at[idx_v.at[li]], gbufs[s], gsems[s])

            def wait_gather(li, s):
                pltpu.make_async_copy(
                    tab.at[idx_v.at[li]], gbufs[s], gsems[s]).wait()

            def fire_wb(li, s):
                pltpu.async_copy(tbufs[s], out.at[li, :, wid], wsems[s])

            def wait_wb(li, s):
                pltpu.make_async_copy(
                    tbufs[s], out.at[li, :, wid], wsems[s]).wait()

            fire_gather(0, 0)
            fire_gather(1, 1)

            def body(p, carry):
                for s in range(2):
                    li = 2 * p + s

                    @pl.when(li >= 2)
                    def _():
                        wait_wb(li - 2, s)

                    wait_gather(li, s)
                    transpose_tile(gbufs[s], tbufs[s])

                    @pl.when(li + 2 < l)
                    def _():
                        fire_gather(li + 2, s)

                    fire_wb(li, s)
                return carry

            lax.fori_loop(0, l // 2, body, 0)
            wait_wb(l - 2, 0)
            wait_wb(l - 1, 1)

        phase(src_tab, sidx_v, src_out)
        phase(tgt_tab, tidx_v, tgt_out)

    return k(src_table, tgt_table, sidx_t, tidx_t)


def kernel(src_table, tgt_table, src_indices, tgt_indices):
    b, l = src_indices.shape
    d = src_table.shape[1]
    sidx_t = jnp.transpose(src_indices.astype(jnp.int32))
    tidx_t = jnp.transpose(tgt_indices.astype(jnp.int32))
    src_phys, tgt_phys = _dual_gather(
        src_table, tgt_table, sidx_t, tidx_t, b, l, d)

    def _relabel(phys):
        return jnp.transpose(phys, (2, 4, 0, 1, 3)).reshape(b, l, d)

    return (_relabel(src_phys), _relabel(tgt_phys))


# speedup vs baseline: 1.8050x; 1.8050x over previous
import functools

import jax
import jax.numpy as jnp
from jax import lax
from jax.experimental import pallas as pl
from jax.experimental.pallas import tpu as pltpu
from jax.experimental.pallas import tpu_sc as plsc

NC, NS = 2, 16
NW = NC * NS
BW = 128


@functools.partial(jax.jit, static_argnums=(4, 5, 6))
def _dual_gather(src_table, tgt_table, sidx_t, tidx_t, b, l, d):
    mesh = plsc.VectorSubcoreMesh(core_axis_name="c", subcore_axis_name="s")

    @functools.partial(
        pl.kernel,
        out_type=(
            jax.ShapeDtypeStruct((l, d // 8, NW, 8, BW), jnp.float32),
            jax.ShapeDtypeStruct((l, d // 8, NW, 8, BW), jnp.float32),
        ),
        mesh=mesh,
        scratch_types=[
            pltpu.VMEM((l, BW), jnp.int32),
            pltpu.VMEM((l, BW), jnp.int32),
            pltpu.VMEM((BW, d), jnp.float32),
            pltpu.VMEM((BW, d), jnp.float32),
            pltpu.VMEM((d // 8, 8, BW), jnp.float32),
            pltpu.VMEM((d // 8, 8, BW), jnp.float32),
            pltpu.SemaphoreType.DMA,
            pltpu.SemaphoreType.DMA,
            pltpu.SemaphoreType.DMA,
            pltpu.SemaphoreType.DMA,
        ],
        compiler_params=pltpu.CompilerParams(
            use_tc_tiling_on_sc=False, needs_layout_passes=False),
    )
    def k(src_tab, tgt_tab, sidx_hbm, tidx_hbm, src_out, tgt_out,
          sidx_v, tidx_v, gbuf0, gbuf1, tbuf0, tbuf1, g0, g1, w0, w1):
        wid = lax.axis_index("s") * NC + lax.axis_index("c")
        col0 = wid * BW
        pltpu.sync_copy(sidx_hbm.at[:, pl.ds(col0, BW)], sidx_v)
        pltpu.sync_copy(tidx_hbm.at[:, pl.ds(col0, BW)], tidx_v)

        lanes = lax.iota(jnp.int32, 16)

        rows16 = [lanes + i0 for i0 in range(0, BW, 16)]

        def transpose_tile(gb, tb):
            @plsc.parallel_loop(0, d, unroll=4)
            def _(e):
                e8 = e // 8
                e1 = e - 8 * e8
                col = jnp.full((16,), 0, jnp.int32) + e
                for i, r in enumerate(rows16):
                    v = plsc.load_gather(gb, [r, col])
                    tb[e8, e1, pl.ds(i * 16, 16)] = v

        gbufs = (gbuf0, gbuf1)
        tbufs = (tbuf0, tbuf1)
        gsems = (g0, g1)
        wsems = (w0, w1)

        def phase(tab, idx_v, out):
            def fire_gather(li, s):
                pltpu.async_copy(tab.at[idx_v.at[li]], gbufs[s], gsems[s])

            def wait_gather(li, s):
                pltpu.make_async_copy(
                    tab.at[idx_v.at[li]], gbufs[s], gsems[s]).wait()

            def fire_wb(li, s):
                pltpu.async_copy(tbufs[s], out.at[li, :, wid], wsems[s])

            def wait_wb(li, s):
                pltpu.make_async_copy(
                    tbufs[s], out.at[li, :, wid], wsems[s]).wait()

            fire_gather(0, 0)
            fire_gather(1, 1)

            def body(p, carry):
                for s in range(2):
                    li = 2 * p + s

                    @pl.when(li >= 2)
                    def _():
                        wait_wb(li - 2, s)

                    wait_gather(li, s)
                    transpose_tile(gbufs[s], tbufs[s])

                    @pl.when(li + 2 < l)
                    def _():
                        fire_gather(li + 2, s)

                    fire_wb(li, s)
                return carry

            lax.fori_loop(0, l // 2, body, 0)
            wait_wb(l - 2, 0)
            wait_wb(l - 1, 1)

        phase(src_tab, sidx_v, src_out)
        phase(tgt_tab, tidx_v, tgt_out)

    return k(src_table, tgt_table, sidx_t, tidx_t)


def kernel(src_table, tgt_table, src_indices, tgt_indices):
    b, l = src_indices.shape
    d = src_table.shape[1]
    sidx_t = jnp.transpose(src_indices.astype(jnp.int32))
    tidx_t = jnp.transpose(tgt_indices.astype(jnp.int32))
    src_phys, tgt_phys = _dual_gather(
        src_table, tgt_table, sidx_t, tidx_t, b, l, d)

    def _relabel(phys):
        return jnp.transpose(phys, (2, 4, 0, 1, 3)).reshape(b, l, d)

    return (_relabel(src_phys), _relabel(tgt_phys))
